# baseline (speedup 1.0000x reference)
import functools

import jax
import jax.numpy as jnp
from jax import lax
from jax.experimental import pallas as pl
from jax.experimental.pallas import tpu as pltpu
from jax.experimental.pallas import tpu_sc as plsc

_EPS = 1e-15
_MAXN = 1.0 - 1e-5
_NC, _NS = 2, 16


def _n2(v):
    return jnp.sum(v * v, axis=-1, keepdims=True)


def _normc(v):
    return jnp.clip(jnp.sqrt(_n2(v)), _EPS, None)


def _artanh(z):
    z = jnp.clip(z, -1 + 1e-7, 1 - 1e-7)
    return 0.5 * jnp.log((1 + z) / (1 - z))


def _proj(v):
    nrm = _normc(v)
    return jnp.where(nrm > _MAXN, v / nrm * _MAXN, v)


def _expmap0(u):
    nrm = _normc(u)
    return jnp.tanh(nrm) * u / nrm


def _logmap0(v):
    nrm = _normc(v)
    return _artanh(nrm) * v / nrm


def _mobius_add(xv, yv):
    x2 = _n2(xv)
    y2 = _n2(yv)
    xy = jnp.sum(xv * yv, axis=-1, keepdims=True)
    num = (1 + 2 * xy + y2) * xv + (1 - x2) * yv
    den = 1 + 2 * xy + x2 * y2
    return num / jnp.clip(den, _EPS, None)


def _precompute(x, kernel_tangents, W, b):
    n, d = x.shape
    K = W.shape[0]
    bn = 1000 if n % 1000 == 0 else n

    def body(x_ref, kt_ref, w_ref, b_ref, t32_ref, t16_ref, xk_ref):
        xv = x_ref[...]
        xh = _proj(_expmap0(xv))
        kp = _proj(_expmap0(kt_ref[...]))
        hb = _proj(_expmap0(b_ref[...]))
        t32_ref[...] = xh
        xn = _normc(xh)
        art = _artanh(xn)
        for k in range(K):
            xkk = _proj(_mobius_add(xh, kp[k][None, :]))
            xk_ref[:, k * d:(k + 1) * d] = xkk
            mx = lax.dot_general(xh, w_ref[k], (((1,), (1,)), ((), ())),
                                 preferred_element_type=jnp.float32)
            mxn = _normc(mx)
            res = _proj(jnp.tanh(mxn / xn * art) * mx / mxn)
            yk = _proj(_mobius_add(res, hb[k][None, :]))
            kkl = 2.0 * yk / (1.0 + _n2(yk))
            gk = 1.0 / jnp.sqrt(jnp.clip(1.0 - _n2(kkl), _EPS, None))
            t16_ref[:, k, :] = (gk * kkl).astype(jnp.bfloat16)

    return pl.pallas_call(
        body,
        grid=(n // bn,),
        in_specs=[
            pl.BlockSpec((bn, d), lambda i: (i, 0)),
            pl.BlockSpec((K, d), lambda i: (0, 0)),
            pl.BlockSpec((K, d, d), lambda i: (0, 0, 0)),
            pl.BlockSpec((K, d), lambda i: (0, 0)),
        ],
        out_specs=[
            pl.BlockSpec((bn, d), lambda i: (i, 0)),
            pl.BlockSpec((bn, K, d), lambda i: (i, 0, 0)),
            pl.BlockSpec((bn, K * d), lambda i: (i, 0)),
        ],
        out_shape=[
            jax.ShapeDtypeStruct((n, d), jnp.float32),
            jax.ShapeDtypeStruct((n, K, d), jnp.bfloat16),
            jax.ShapeDtypeStruct((n, K * d), jnp.float32),
        ],
    )(x, kernel_tangents, W, b)


def _sc_gather(nei_flat, t32, t16bits):
    e = nei_flat.shape[0]
    d = t32.shape[1]
    h = t16bits.shape[1]
    nw = _NC * _NS
    bpw = e // nw
    ch = 40
    assert e % nw == 0 and bpw % ch == 0 and (bpw % 8) == 0
    mesh = plsc.VectorSubcoreMesh(core_axis_name="c", subcore_axis_name="s",
                                  num_cores=_NC, num_subcores=_NS)

    nch = bpw // ch

    @functools.partial(
        pl.kernel,
        out_type=[
            jax.ShapeDtypeStruct((e, d), jnp.float32),
            jax.ShapeDtypeStruct((e, h), jnp.int32),
        ],
        mesh=mesh,
        scratch_types=[
            pltpu.VMEM((2, ch), jnp.int32),
            pltpu.VMEM((2, ch, d), jnp.float32),
            pltpu.VMEM((2, ch, h), jnp.int32),
            pltpu.SemaphoreType.DMA((2,)),
            pltpu.SemaphoreType.DMA((2,)),
        ],
    )
    def gather_k(nei_hbm, t32_hbm, t16_hbm, o32_hbm, o16_hbm,
                 idx_v, r32_v, r16_v, sem32, sem16):
        wid = lax.axis_index("s") * _NC + lax.axis_index("c")
        base = wid * bpw

        def start(t, s):
            pltpu.sync_copy(nei_hbm.at[pl.ds(base + t * ch, ch)],
                            idx_v.at[s])
            pltpu.async_copy(t32_hbm.at[idx_v.at[s]], r32_v.at[s],
                             sem32.at[s])
            pltpu.async_copy(t16_hbm.at[idx_v.at[s]], r16_v.at[s],
                             sem16.at[s])

        start(0, 0)

        def chunk(t, carry):
            s = t % 2
            @pl.when(t + 1 < nch)
            def _():
                start(t + 1, 1 - s)

            pltpu.make_async_copy(t32_hbm.at[idx_v.at[s]], r32_v.at[s],
                                  sem32.at[s]).wait()
            pltpu.make_async_copy(t16_hbm.at[idx_v.at[s]], r16_v.at[s],
                                  sem16.at[s]).wait()
            pltpu.sync_copy(r32_v.at[s], o32_hbm.at[pl.ds(base + t * ch, ch)])
            pltpu.sync_copy(r16_v.at[s], o16_hbm.at[pl.ds(base + t * ch, ch)])
            return carry

        lax.fori_loop(0, nch, chunk, 0)

    return gather_k(nei_flat, t32, t16bits)


def _aggregate(G32, G16, XK, nei_mask):
    n, deg = nei_mask.shape
    d = G32.shape[1]
    K = G16.shape[1]
    bn = 200 if n % 200 == 0 else n
    e2 = bn * deg

    def body(g_ref, g16_ref, xk_ref, m_ref, o_ref):
        iota_r = lax.broadcasted_iota(jnp.int32, (deg * d, deg), 0)
        iota_c = lax.broadcasted_iota(jnp.int32, (deg * d, deg), 1)
        bcol = (iota_r // d == iota_c).astype(jnp.float32)
        brow = (lax.broadcasted_iota(jnp.int32, (deg, deg * d), 0)
                == lax.broadcasted_iota(jnp.int32, (deg, deg * d), 1) // d
                ).astype(jnp.float32)
        ones_dd = jnp.ones((d, deg), jnp.float32)
        dn = (((1,), (0,)), ((), ()))

        def reduce_lanes(p3):
            return lax.dot_general(p3.reshape(bn, deg * d), bcol, dn,
                                   preferred_element_type=jnp.float32)

        def expand_lanes(s2):
            return lax.dot_general(s2, brow, dn,
                                   preferred_element_type=jnp.float32
                                   ).reshape(bn, deg, d)

        xh = g_ref[...].reshape(bn, deg, d)
        mask = m_ref[...]
        x2 = reduce_lanes(xh * xh)
        bc = 1.0 - x2
        num = jnp.zeros((bn, deg, d), jnp.float32)
        den = jnp.zeros((bn, deg), jnp.float32)
        for k in range(K):
            xkk = xk_ref[:, k * d:(k + 1) * d]
            y2 = lax.dot_general(xkk * xkk, ones_dd, dn,
                                 preferred_element_type=jnp.float32)
            xy = reduce_lanes(xh * xkk[:, None, :])
            av = 1.0 - 2.0 * xy + y2
            dden = jnp.clip(1.0 - 2.0 * xy + x2 * y2, _EPS, None)
            nsq = jnp.clip(av * av * x2 - 2.0 * av * bc * xy + bc * bc * y2,
                           0.0, None)
            z = jnp.clip(jnp.sqrt(nsq) / dden, 0.0, 1 - 1e-5)
            dk = jnp.log((1 + z) / (1 - z))
            wk = jnp.maximum(1.0 - dk, 0.0) * mask
            gkk = g16_ref[:, k, :].astype(jnp.float32).reshape(bn, deg, d)
            gsrc = jnp.sqrt(1.0 + reduce_lanes(gkk * gkk))
            num = num + expand_lanes(wk) * gkk
            den = den + wk * gsrc
        denc = jnp.clip(den, 1e-9, None)
        nn = reduce_lanes(num * num)
        g2 = 1.0 / jnp.sqrt(jnp.clip(1.0 - nn / (denc * denc), _EPS, None))
        mg = mask * g2
        aggn = jnp.sum(expand_lanes(mg / denc) * num, axis=1)
        aggd = jnp.clip(jnp.sum(mg, axis=-1, keepdims=True), 1e-9, None)
        agg = aggn / aggd
        a2 = _n2(agg)
        res = agg / (1.0 + jnp.sqrt(jnp.clip(1.0 - a2, _EPS, None)))
        res = _proj(res)
        res = _proj(_expmap0(jax.nn.relu(_logmap0(res))))
        o_ref[...] = res

    return pl.pallas_call(
        body,
        grid=(n // bn,),
        in_specs=[
            pl.BlockSpec((e2, d), lambda i: (i, 0)),
            pl.BlockSpec((e2, K, d), lambda i: (i, 0, 0)),
            pl.BlockSpec((bn, XK.shape[1]), lambda i: (i, 0)),
            pl.BlockSpec((bn, deg), lambda i: (i, 0)),
        ],
        out_specs=pl.BlockSpec((bn, d), lambda i: (i, 0)),
        out_shape=jax.ShapeDtypeStruct((n, d), jnp.float32),
    )(G32, G16, XK, nei_mask)


def kernel(x, nei, nei_mask, kernel_tangents, W, b):
    n, deg = nei.shape
    T32, T16, XK = _precompute(x, kernel_tangents, W, b)
    K, dd = T16.shape[1], T16.shape[2]
    T16bits = lax.bitcast_convert_type(T16.reshape(n, K * dd // 2, 2),
                                       jnp.int32)
    nei_flat = nei.reshape(-1).astype(jnp.int32)
    mask = nei_mask.astype(jnp.float32)
    ns = 5
    rows = n // ns
    outs = []
    for s in range(ns):
        lo = s * rows
        g32_s, g16b_s = _sc_gather(
            lax.dynamic_slice_in_dim(nei_flat, lo * deg, rows * deg),
            T32, T16bits)
        g16_s = lax.bitcast_convert_type(
            g16b_s, jnp.bfloat16).reshape(rows * deg, K, dd)
        outs.append(_aggregate(
            g32_s, g16_s,
            lax.dynamic_slice_in_dim(XK, lo, rows),
            lax.dynamic_slice_in_dim(mask, lo, rows)))
    return jnp.concatenate(outs, axis=0)


# speedup vs baseline: 2.5424x; 2.5424x over previous
import functools

import jax
import jax.numpy as jnp
from jax import lax
from jax.experimental import pallas as pl
from jax.experimental.pallas import tpu as pltpu
from jax.experimental.pallas import tpu_sc as plsc

_EPS = 1e-15
_MAXN = 1.0 - 1e-5
_NC, _NS = 2, 16


def _n2(v):
    return jnp.sum(v * v, axis=-1, keepdims=True)


def _normc(v):
    return jnp.clip(jnp.sqrt(_n2(v)), _EPS, None)


def _artanh(z):
    z = jnp.clip(z, -1 + 1e-7, 1 - 1e-7)
    return 0.5 * jnp.log((1 + z) / (1 - z))


def _proj(v):
    nrm = _normc(v)
    return jnp.where(nrm > _MAXN, v / nrm * _MAXN, v)


def _expmap0(u):
    nrm = _normc(u)
    return jnp.tanh(nrm) * u / nrm


def _logmap0(v):
    nrm = _normc(v)
    return _artanh(nrm) * v / nrm


def _mobius_add(xv, yv):
    x2 = _n2(xv)
    y2 = _n2(yv)
    xy = jnp.sum(xv * yv, axis=-1, keepdims=True)
    num = (1 + 2 * xy + y2) * xv + (1 - x2) * yv
    den = 1 + 2 * xy + x2 * y2
    return num / jnp.clip(den, _EPS, None)


def _precompute(x, kernel_tangents, W, b):
    n, d = x.shape
    K = W.shape[0]
    bn = 1000 if n % 1000 == 0 else n

    def body(x_ref, kt_ref, w_ref, b_ref, t32_ref, t16_ref, xk_ref):
        xv = x_ref[...]
        xh = _proj(_expmap0(xv))
        kp = _proj(_expmap0(kt_ref[...]))
        hb = _proj(_expmap0(b_ref[...]))
        t32_ref[...] = xh
        xn = _normc(xh)
        art = _artanh(xn)
        for k in range(K):
            xkk = _proj(_mobius_add(xh, kp[k][None, :]))
            xk_ref[:, k * d:(k + 1) * d] = xkk
            mx = lax.dot_general(xh, w_ref[k], (((1,), (1,)), ((), ())),
                                 preferred_element_type=jnp.float32)
            mxn = _normc(mx)
            res = _proj(jnp.tanh(mxn / xn * art) * mx / mxn)
            yk = _proj(_mobius_add(res, hb[k][None, :]))
            kkl = 2.0 * yk / (1.0 + _n2(yk))
            gk = 1.0 / jnp.sqrt(jnp.clip(1.0 - _n2(kkl), _EPS, None))
            t16_ref[:, k, :] = gk * kkl

    return pl.pallas_call(
        body,
        grid=(n // bn,),
        in_specs=[
            pl.BlockSpec((bn, d), lambda i: (i, 0)),
            pl.BlockSpec((K, d), lambda i: (0, 0)),
            pl.BlockSpec((K, d, d), lambda i: (0, 0, 0)),
            pl.BlockSpec((K, d), lambda i: (0, 0)),
        ],
        out_specs=[
            pl.BlockSpec((bn, d), lambda i: (i, 0)),
            pl.BlockSpec((bn, K, d), lambda i: (i, 0, 0)),
            pl.BlockSpec((bn, K * d), lambda i: (i, 0)),
        ],
        out_shape=[
            jax.ShapeDtypeStruct((n, d), jnp.float32),
            jax.ShapeDtypeStruct((n, K, d), jnp.float32),
            jax.ShapeDtypeStruct((n, K * d), jnp.float32),
        ],
    )(x, kernel_tangents, W, b)


def _sc_gather(nei_flat, t32, t4):
    e = nei_flat.shape[0]
    d = t32.shape[1]
    K = t4.shape[1]
    nw = _NC * _NS
    bpw = e // nw
    ch = 40
    assert e % nw == 0 and bpw % ch == 0 and (bpw % 8) == 0
    mesh = plsc.VectorSubcoreMesh(core_axis_name="c", subcore_axis_name="s",
                                  num_cores=_NC, num_subcores=_NS)

    nch = bpw // ch

    @functools.partial(
        pl.kernel,
        out_type=[
            jax.ShapeDtypeStruct((e, d), jnp.float32),
            jax.ShapeDtypeStruct((e, K, d), jnp.float32),
        ],
        mesh=mesh,
        scratch_types=[
            pltpu.VMEM((2, ch), jnp.int32),
            pltpu.VMEM((2, ch, d), jnp.float32),
            pltpu.VMEM((2, ch, K, d), jnp.float32),
            pltpu.SemaphoreType.DMA((2,)),
            pltpu.SemaphoreType.DMA((2,)),
        ],
    )
    def gather_k(nei_hbm, t32_hbm, t16_hbm, o32_hbm, o16_hbm,
                 idx_v, r32_v, r16_v, sem32, sem16):
        wid = lax.axis_index("s") * _NC + lax.axis_index("c")
        base = wid * bpw

        def start(t, s):
            pltpu.sync_copy(nei_hbm.at[pl.ds(base + t * ch, ch)],
                            idx_v.at[s])
            pltpu.async_copy(t32_hbm.at[idx_v.at[s]], r32_v.at[s],
                             sem32.at[s])
            pltpu.async_copy(t16_hbm.at[idx_v.at[s]], r16_v.at[s],
                             sem16.at[s])

        start(0, 0)

        def chunk(t, carry):
            s = t % 2
            @pl.when(t + 1 < nch)
            def _():
                start(t + 1, 1 - s)

            pltpu.make_async_copy(t32_hbm.at[idx_v.at[s]], r32_v.at[s],
                                  sem32.at[s]).wait()
            pltpu.make_async_copy(t16_hbm.at[idx_v.at[s]], r16_v.at[s],
                                  sem16.at[s]).wait()
            pltpu.sync_copy(r32_v.at[s], o32_hbm.at[pl.ds(base + t * ch, ch)])
            pltpu.sync_copy(r16_v.at[s], o16_hbm.at[pl.ds(base + t * ch, ch)])
            return carry

        lax.fori_loop(0, nch, chunk, 0)

    return gather_k(nei_flat, t32, t4)


def _aggregate(G32, G16, XK, nei_mask):
    n, deg = nei_mask.shape
    d = G32.shape[1]
    K = G16.shape[1]
    bn = 200 if n % 200 == 0 else n
    e2 = bn * deg

    def body(g_ref, g16_ref, xk_ref, m_ref, o_ref):
        iota_r = lax.broadcasted_iota(jnp.int32, (deg * d, deg), 0)
        iota_c = lax.broadcasted_iota(jnp.int32, (deg * d, deg), 1)
        bcol = (iota_r // d == iota_c).astype(jnp.float32)
        brow = (lax.broadcasted_iota(jnp.int32, (deg, deg * d), 0)
                == lax.broadcasted_iota(jnp.int32, (deg, deg * d), 1) // d
                ).astype(jnp.float32)
        ones_dd = jnp.ones((d, deg), jnp.float32)
        dn = (((1,), (0,)), ((), ()))

        def reduce_lanes(p3):
            return lax.dot_general(p3.reshape(bn, deg * d), bcol, dn,
                                   preferred_element_type=jnp.float32)

        def expand_lanes(s2):
            return lax.dot_general(s2, brow, dn,
                                   preferred_element_type=jnp.float32
                                   ).reshape(bn, deg, d)

        xh = g_ref[...].reshape(bn, deg, d)
        mask = m_ref[...]
        x2 = reduce_lanes(xh * xh)
        bc = 1.0 - x2
        num = jnp.zeros((bn, deg, d), jnp.float32)
        den = jnp.zeros((bn, deg), jnp.float32)
        for k in range(K):
            xkk = xk_ref[:, k * d:(k + 1) * d]
            y2 = lax.dot_general(xkk * xkk, ones_dd, dn,
                                 preferred_element_type=jnp.float32)
            xy = reduce_lanes(xh * xkk[:, None, :])
            av = 1.0 - 2.0 * xy + y2
            dden = jnp.clip(1.0 - 2.0 * xy + x2 * y2, _EPS, None)
            nsq = jnp.clip(av * av * x2 - 2.0 * av * bc * xy + bc * bc * y2,
                           0.0, None)
            z = jnp.clip(jnp.sqrt(nsq) / dden, 0.0, 1 - 1e-5)
            dk = jnp.log((1 + z) / (1 - z))
            wk = jnp.maximum(1.0 - dk, 0.0) * mask
            gkk = g16_ref[:, k, :].reshape(bn, deg, d)
            gsrc = jnp.sqrt(1.0 + reduce_lanes(gkk * gkk))
            num = num + expand_lanes(wk) * gkk
            den = den + wk * gsrc
        denc = jnp.clip(den, 1e-9, None)
        nn = reduce_lanes(num * num)
        g2 = 1.0 / jnp.sqrt(jnp.clip(1.0 - nn / (denc * denc), _EPS, None))
        mg = mask * g2
        aggn = jnp.sum(expand_lanes(mg / denc) * num, axis=1)
        aggd = jnp.clip(jnp.sum(mg, axis=-1, keepdims=True), 1e-9, None)
        agg = aggn / aggd
        a2 = _n2(agg)
        res = agg / (1.0 + jnp.sqrt(jnp.clip(1.0 - a2, _EPS, None)))
        res = _proj(res)
        res = _proj(_expmap0(jax.nn.relu(_logmap0(res))))
        o_ref[...] = res

    return pl.pallas_call(
        body,
        grid=(n // bn,),
        in_specs=[
            pl.BlockSpec((e2, d), lambda i: (i, 0)),
            pl.BlockSpec((e2, K, d), lambda i: (i, 0, 0)),
            pl.BlockSpec((bn, XK.shape[1]), lambda i: (i, 0)),
            pl.BlockSpec((bn, deg), lambda i: (i, 0)),
        ],
        out_specs=pl.BlockSpec((bn, d), lambda i: (i, 0)),
        out_shape=jax.ShapeDtypeStruct((n, d), jnp.float32),
    )(G32, G16, XK, nei_mask)


def kernel(x, nei, nei_mask, kernel_tangents, W, b):
    n, deg = nei.shape
    T32, T4, XK = _precompute(x, kernel_tangents, W, b)
    nei_flat = nei.reshape(-1).astype(jnp.int32)
    mask = nei_mask.astype(jnp.float32)
    ns = 5
    rows = n // ns
    outs = []
    for s in range(ns):
        lo = s * rows
        g32_s, g4_s = _sc_gather(
            lax.dynamic_slice_in_dim(nei_flat, lo * deg, rows * deg),
            T32, T4)
        outs.append(_aggregate(
            g32_s, g4_s,
            lax.dynamic_slice_in_dim(XK, lo, rows),
            lax.dynamic_slice_in_dim(mask, lo, rows)))
    return jnp.concatenate(outs, axis=0)


# speedup vs baseline: 4.5818x; 1.8022x over previous
import functools

import jax
import jax.numpy as jnp
from jax import lax
from jax.experimental import pallas as pl
from jax.experimental.pallas import tpu as pltpu
from jax.experimental.pallas import tpu_sc as plsc

_EPS = 1e-15
_MAXN = 1.0 - 1e-5
_NC, _NS = 2, 16


def _n2(v):
    return jnp.sum(v * v, axis=-1, keepdims=True)


def _normc(v):
    return jnp.clip(jnp.sqrt(_n2(v)), _EPS, None)


def _artanh(z):
    z = jnp.clip(z, -1 + 1e-7, 1 - 1e-7)
    return 0.5 * jnp.log((1 + z) / (1 - z))


def _proj(v):
    nrm = _normc(v)
    return jnp.where(nrm > _MAXN, v / nrm * _MAXN, v)


def _expmap0(u):
    nrm = _normc(u)
    return jnp.tanh(nrm) * u / nrm


def _logmap0(v):
    nrm = _normc(v)
    return _artanh(nrm) * v / nrm


def _mobius_add(xv, yv):
    x2 = _n2(xv)
    y2 = _n2(yv)
    xy = jnp.sum(xv * yv, axis=-1, keepdims=True)
    num = (1 + 2 * xy + y2) * xv + (1 - x2) * yv
    den = 1 + 2 * xy + x2 * y2
    return num / jnp.clip(den, _EPS, None)


def _precompute(x, kernel_tangents, W, b):
    n, d = x.shape
    K = W.shape[0]
    bn = 1000 if n % 1000 == 0 else n

    def body(x_ref, kt_ref, w_ref, b_ref, t_ref, xk_ref):
        xv = x_ref[...]
        xh = _proj(_expmap0(xv))
        kp = _proj(_expmap0(kt_ref[...]))
        hb = _proj(_expmap0(b_ref[...]))
        t_ref[:, 0:d] = xh
        xn = _normc(xh)
        art = _artanh(xn)
        for k in range(K):
            xkk = _proj(_mobius_add(xh, kp[k][None, :]))
            xk_ref[:, k * d:(k + 1) * d] = xkk
            mx = lax.dot_general(xh, w_ref[k], (((1,), (1,)), ((), ())),
                                 preferred_element_type=jnp.float32)
            mxn = _normc(mx)
            res = _proj(jnp.tanh(mxn / xn * art) * mx / mxn)
            yk = _proj(_mobius_add(res, hb[k][None, :]))
            kkl = 2.0 * yk / (1.0 + _n2(yk))
            gk = 1.0 / jnp.sqrt(jnp.clip(1.0 - _n2(kkl), _EPS, None))
            t_ref[:, (k + 1) * d:(k + 2) * d] = gk * kkl

    return pl.pallas_call(
        body,
        grid=(n // bn,),
        in_specs=[
            pl.BlockSpec((bn, d), lambda i: (i, 0)),
            pl.BlockSpec((K, d), lambda i: (0, 0)),
            pl.BlockSpec((K, d, d), lambda i: (0, 0, 0)),
            pl.BlockSpec((K, d), lambda i: (0, 0)),
        ],
        out_specs=[
            pl.BlockSpec((bn, (1 + K) * d), lambda i: (i, 0)),
            pl.BlockSpec((bn, K * d), lambda i: (i, 0)),
        ],
        out_shape=[
            jax.ShapeDtypeStruct((n, (1 + K) * d), jnp.float32),
            jax.ShapeDtypeStruct((n, K * d), jnp.float32),
        ],
    )(x, kernel_tangents, W, b)


def _sc_gather(nei_flat, table):
    e = nei_flat.shape[0]
    row = table.shape[1]
    npart = row // 128
    nw = _NC * _NS
    bpw = e // nw
    ch = 40
    assert e % nw == 0 and bpw % ch == 0 and (bpw % 8) == 0
    mesh = plsc.VectorSubcoreMesh(core_axis_name="c", subcore_axis_name="s",
                                  num_cores=_NC, num_subcores=_NS)

    nch = bpw // ch

    @functools.partial(
        pl.kernel,
        out_type=[jax.ShapeDtypeStruct((e, 128), jnp.float32)
                  for _ in range(npart)],
        mesh=mesh,
        scratch_types=[
            pltpu.VMEM((2, ch), jnp.int32),
            pltpu.VMEM((2, ch, row), jnp.float32),
            pltpu.SemaphoreType.DMA((2,)),
        ],
    )
    def gather_k(nei_hbm, table_hbm, *rest):
        outs = rest[:npart]
        idx_v, rows_v, sem = rest[npart:]
        wid = lax.axis_index("s") * _NC + lax.axis_index("c")
        base = wid * bpw

        def start(t, s):
            pltpu.sync_copy(nei_hbm.at[pl.ds(base + t * ch, ch)],
                            idx_v.at[s])
            pltpu.async_copy(table_hbm.at[idx_v.at[s]], rows_v.at[s],
                             sem.at[s])

        start(0, 0)

        def chunk(t, carry):
            s = t % 2
            @pl.when(t + 1 < nch)
            def _():
                start(t + 1, 1 - s)

            pltpu.make_async_copy(table_hbm.at[idx_v.at[s]], rows_v.at[s],
                                  sem.at[s]).wait()
            for j in range(npart):
                pltpu.sync_copy(rows_v.at[s, :, pl.ds(j * 128, 128)],
                                outs[j].at[pl.ds(base + t * ch, ch)])
            return carry

        lax.fori_loop(0, nch, chunk, 0)

    return gather_k(nei_flat, table)


def _aggregate(Gparts, XK, nei_mask):
    n, deg = nei_mask.shape
    d = Gparts[0].shape[1]
    K = len(Gparts) - 1
    bn = 200 if n % 200 == 0 else n
    e2 = bn * deg

    def body(g_ref, gk0, gk1, gk2, gk3, xk_ref, m_ref, o_ref):
        gks = (gk0, gk1, gk2, gk3)
        iota_r = lax.broadcasted_iota(jnp.int32, (deg * d, deg), 0)
        iota_c = lax.broadcasted_iota(jnp.int32, (deg * d, deg), 1)
        bcol = (iota_r // d == iota_c).astype(jnp.float32)
        brow = (lax.broadcasted_iota(jnp.int32, (deg, deg * d), 0)
                == lax.broadcasted_iota(jnp.int32, (deg, deg * d), 1) // d
                ).astype(jnp.float32)
        ones_dd = jnp.ones((d, deg), jnp.float32)
        dn = (((1,), (0,)), ((), ()))

        def reduce_lanes(p3):
            return lax.dot_general(p3.reshape(bn, deg * d), bcol, dn,
                                   preferred_element_type=jnp.float32)

        def expand_lanes(s2):
            return lax.dot_general(s2, brow, dn,
                                   preferred_element_type=jnp.float32
                                   ).reshape(bn, deg, d)

        xh = g_ref[...].reshape(bn, deg, d)
        mask = m_ref[...]
        x2 = reduce_lanes(xh * xh)
        bc = 1.0 - x2
        num = jnp.zeros((bn, deg, d), jnp.float32)
        den = jnp.zeros((bn, deg), jnp.float32)
        for k in range(K):
            xkk = xk_ref[:, k * d:(k + 1) * d]
            y2 = lax.dot_general(xkk * xkk, ones_dd, dn,
                                 preferred_element_type=jnp.float32)
            xy = reduce_lanes(xh * xkk[:, None, :])
            av = 1.0 - 2.0 * xy + y2
            dden = jnp.clip(1.0 - 2.0 * xy + x2 * y2, _EPS, None)
            nsq = jnp.clip(av * av * x2 - 2.0 * av * bc * xy + bc * bc * y2,
                           0.0, None)
            z = jnp.clip(jnp.sqrt(nsq) / dden, 0.0, 1 - 1e-5)
            dk = jnp.log((1 + z) / (1 - z))
            wk = jnp.maximum(1.0 - dk, 0.0) * mask
            gkk = gks[k][...].reshape(bn, deg, d)
            gsrc = jnp.sqrt(1.0 + reduce_lanes(gkk * gkk))
            num = num + expand_lanes(wk) * gkk
            den = den + wk * gsrc
        denc = jnp.clip(den, 1e-9, None)
        nn = reduce_lanes(num * num)
        g2 = 1.0 / jnp.sqrt(jnp.clip(1.0 - nn / (denc * denc), _EPS, None))
        mg = mask * g2
        aggn = jnp.sum(expand_lanes(mg / denc) * num, axis=1)
        aggd = jnp.clip(jnp.sum(mg, axis=-1, keepdims=True), 1e-9, None)
        agg = aggn / aggd
        a2 = _n2(agg)
        res = agg / (1.0 + jnp.sqrt(jnp.clip(1.0 - a2, _EPS, None)))
        res = _proj(res)
        res = _proj(_expmap0(jax.nn.relu(_logmap0(res))))
        o_ref[...] = res

    return pl.pallas_call(
        body,
        grid=(n // bn,),
        in_specs=(
            [pl.BlockSpec((e2, d), lambda i: (i, 0))
             for _ in range(1 + K)]
            + [pl.BlockSpec((bn, XK.shape[1]), lambda i: (i, 0)),
               pl.BlockSpec((bn, deg), lambda i: (i, 0))]
        ),
        out_specs=pl.BlockSpec((bn, d), lambda i: (i, 0)),
        out_shape=jax.ShapeDtypeStruct((n, d), jnp.float32),
    )(*Gparts, XK, nei_mask)


def kernel(x, nei, nei_mask, kernel_tangents, W, b):
    n, deg = nei.shape
    T, XK = _precompute(x, kernel_tangents, W, b)
    nei_flat = nei.reshape(-1).astype(jnp.int32)
    mask = nei_mask.astype(jnp.float32)
    ns = 5
    rows = n // ns
    outs = []
    for s in range(ns):
        lo = s * rows
        gparts = _sc_gather(
            lax.dynamic_slice_in_dim(nei_flat, lo * deg, rows * deg), T)
        outs.append(_aggregate(
            gparts,
            lax.dynamic_slice_in_dim(XK, lo, rows),
            lax.dynamic_slice_in_dim(mask, lo, rows)))
    return jnp.concatenate(outs, axis=0)


# speedup vs baseline: 4.8517x; 1.0589x over previous
import functools

import jax
import jax.numpy as jnp
from jax import lax
from jax.experimental import pallas as pl
from jax.experimental.pallas import tpu as pltpu
from jax.experimental.pallas import tpu_sc as plsc

_EPS = 1e-15
_MAXN = 1.0 - 1e-5
_NC, _NS = 2, 16


def _n2(v):
    return jnp.sum(v * v, axis=-1, keepdims=True)


def _normc(v):
    return jnp.clip(jnp.sqrt(_n2(v)), _EPS, None)


def _artanh(z):
    z = jnp.clip(z, -1 + 1e-7, 1 - 1e-7)
    return 0.5 * jnp.log((1 + z) / (1 - z))


def _proj(v):
    nrm = _normc(v)
    return jnp.where(nrm > _MAXN, v / nrm * _MAXN, v)


def _expmap0(u):
    nrm = _normc(u)
    return jnp.tanh(nrm) * u / nrm


def _logmap0(v):
    nrm = _normc(v)
    return _artanh(nrm) * v / nrm


def _mobius_add(xv, yv):
    x2 = _n2(xv)
    y2 = _n2(yv)
    xy = jnp.sum(xv * yv, axis=-1, keepdims=True)
    num = (1 + 2 * xy + y2) * xv + (1 - x2) * yv
    den = 1 + 2 * xy + x2 * y2
    return num / jnp.clip(den, _EPS, None)


def _precompute(x, kernel_tangents, W, b):
    n, d = x.shape
    K = W.shape[0]
    bn = 1000 if n % 1000 == 0 else n

    def body(x_ref, kt_ref, w_ref, b_ref, t_ref, xk_ref):
        xv = x_ref[...]
        xh = _proj(_expmap0(xv))
        kp = _proj(_expmap0(kt_ref[...]))
        hb = _proj(_expmap0(b_ref[...]))
        t_ref[:, 0:d] = xh
        xn = _normc(xh)
        art = _artanh(xn)
        gkks = []
        for k in range(K):
            xkk = _proj(_mobius_add(xh, kp[k][None, :]))
            xk_ref[:, k * d:(k + 1) * d] = xkk
            mx = lax.dot_general(xh, w_ref[k], (((1,), (1,)), ((), ())),
                                 preferred_element_type=jnp.float32)
            mxn = _normc(mx)
            res = _proj(jnp.tanh(mxn / xn * art) * mx / mxn)
            yk = _proj(_mobius_add(res, hb[k][None, :]))
            kkl = 2.0 * yk / (1.0 + _n2(yk))
            gk = 1.0 / jnp.sqrt(jnp.clip(1.0 - _n2(kkl), _EPS, None))
            gkks.append(gk * kkl)
        for p in range(K // 2):
            lo = lax.bitcast_convert_type(
                gkks[2 * p].astype(jnp.bfloat16), jnp.uint16
            ).astype(jnp.uint32)
            hi = lax.bitcast_convert_type(
                gkks[2 * p + 1].astype(jnp.bfloat16), jnp.uint16
            ).astype(jnp.uint32)
            packed = lo | (hi << 16)
            t_ref[:, (1 + p) * d:(2 + p) * d] = lax.bitcast_convert_type(
                packed, jnp.float32)

    return pl.pallas_call(
        body,
        grid=(n // bn,),
        in_specs=[
            pl.BlockSpec((bn, d), lambda i: (i, 0)),
            pl.BlockSpec((K, d), lambda i: (0, 0)),
            pl.BlockSpec((K, d, d), lambda i: (0, 0, 0)),
            pl.BlockSpec((K, d), lambda i: (0, 0)),
        ],
        out_specs=[
            pl.BlockSpec((bn, (1 + K // 2) * d), lambda i: (i, 0)),
            pl.BlockSpec((bn, K * d), lambda i: (i, 0)),
        ],
        out_shape=[
            jax.ShapeDtypeStruct((n, (1 + K // 2) * d), jnp.float32),
            jax.ShapeDtypeStruct((n, K * d), jnp.float32),
        ],
    )(x, kernel_tangents, W, b)


def _sc_gather(nei_flat, table):
    e = nei_flat.shape[0]
    row = table.shape[1]
    npart = row // 128
    nw = _NC * _NS
    bpw = e // nw
    ch = 40
    assert e % nw == 0 and bpw % ch == 0 and (bpw % 8) == 0
    mesh = plsc.VectorSubcoreMesh(core_axis_name="c", subcore_axis_name="s",
                                  num_cores=_NC, num_subcores=_NS)

    nch = bpw // ch

    @functools.partial(
        pl.kernel,
        out_type=[jax.ShapeDtypeStruct((e, 128), jnp.float32)
                  for _ in range(npart)],
        mesh=mesh,
        scratch_types=[
            pltpu.VMEM((2, ch), jnp.int32),
            pltpu.VMEM((2, ch, row), jnp.float32),
            pltpu.SemaphoreType.DMA((2,)),
        ],
    )
    def gather_k(nei_hbm, table_hbm, *rest):
        outs = rest[:npart]
        idx_v, rows_v, sem = rest[npart:]
        wid = lax.axis_index("s") * _NC + lax.axis_index("c")
        base = wid * bpw

        def start(t, s):
            pltpu.sync_copy(nei_hbm.at[pl.ds(base + t * ch, ch)],
                            idx_v.at[s])
            pltpu.async_copy(table_hbm.at[idx_v.at[s]], rows_v.at[s],
                             sem.at[s])

        start(0, 0)

        def chunk(t, carry):
            s = t % 2
            @pl.when(t + 1 < nch)
            def _():
                start(t + 1, 1 - s)

            pltpu.make_async_copy(table_hbm.at[idx_v.at[s]], rows_v.at[s],
                                  sem.at[s]).wait()
            for j in range(npart):
                pltpu.sync_copy(rows_v.at[s, :, pl.ds(j * 128, 128)],
                                outs[j].at[pl.ds(base + t * ch, ch)])
            return carry

        lax.fori_loop(0, nch, chunk, 0)

    return gather_k(nei_flat, table)


def _aggregate(Gparts, XK, nei_mask):
    n, deg = nei_mask.shape
    d = Gparts[0].shape[1]
    K = XK.shape[1] // d
    bn = 200 if n % 200 == 0 else n
    e2 = bn * deg

    def body(g_ref, p01_ref, p23_ref, xk_ref, m_ref, o_ref):
        slabs = (p01_ref, p23_ref)
        iota_r = lax.broadcasted_iota(jnp.int32, (deg * d, deg), 0)
        iota_c = lax.broadcasted_iota(jnp.int32, (deg * d, deg), 1)
        bcol = (iota_r // d == iota_c).astype(jnp.float32)
        brow = (lax.broadcasted_iota(jnp.int32, (deg, deg * d), 0)
                == lax.broadcasted_iota(jnp.int32, (deg, deg * d), 1) // d
                ).astype(jnp.float32)
        ones_dd = jnp.ones((d, deg), jnp.float32)
        dn = (((1,), (0,)), ((), ()))

        def reduce_lanes(p3):
            return lax.dot_general(p3.reshape(bn, deg * d), bcol, dn,
                                   preferred_element_type=jnp.float32)

        def expand_lanes(s2):
            return lax.dot_general(s2, brow, dn,
                                   preferred_element_type=jnp.float32
                                   ).reshape(bn, deg, d)

        xh = g_ref[...].reshape(bn, deg, d)
        mask = m_ref[...]
        x2 = reduce_lanes(xh * xh)
        bc = 1.0 - x2
        num = jnp.zeros((bn, deg, d), jnp.float32)
        den = jnp.zeros((bn, deg), jnp.float32)
        for k in range(K):
            xkk = xk_ref[:, k * d:(k + 1) * d]
            y2 = lax.dot_general(xkk * xkk, ones_dd, dn,
                                 preferred_element_type=jnp.float32)
            xy = reduce_lanes(xh * xkk[:, None, :])
            av = 1.0 - 2.0 * xy + y2
            dden = jnp.clip(1.0 - 2.0 * xy + x2 * y2, _EPS, None)
            nsq = jnp.clip(av * av * x2 - 2.0 * av * bc * xy + bc * bc * y2,
                           0.0, None)
            z = jnp.clip(jnp.sqrt(nsq) / dden, 0.0, 1 - 1e-5)
            dk = jnp.log((1 + z) / (1 - z))
            wk = jnp.maximum(1.0 - dk, 0.0) * mask
            bits = lax.bitcast_convert_type(slabs[k // 2][...], jnp.uint32)
            gbits = (bits << 16) if k % 2 == 0 else (bits & jnp.uint32(0xFFFF0000))
            gkk = lax.bitcast_convert_type(gbits, jnp.float32
                                           ).reshape(bn, deg, d)
            gsrc = jnp.sqrt(1.0 + reduce_lanes(gkk * gkk))
            num = num + expand_lanes(wk) * gkk
            den = den + wk * gsrc
        denc = jnp.clip(den, 1e-9, None)
        nn = reduce_lanes(num * num)
        g2 = 1.0 / jnp.sqrt(jnp.clip(1.0 - nn / (denc * denc), _EPS, None))
        mg = mask * g2
        aggn = jnp.sum(expand_lanes(mg / denc) * num, axis=1)
        aggd = jnp.clip(jnp.sum(mg, axis=-1, keepdims=True), 1e-9, None)
        agg = aggn / aggd
        a2 = _n2(agg)
        res = agg / (1.0 + jnp.sqrt(jnp.clip(1.0 - a2, _EPS, None)))
        res = _proj(res)
        res = _proj(_expmap0(jax.nn.relu(_logmap0(res))))
        o_ref[...] = res

    return pl.pallas_call(
        body,
        grid=(n // bn,),
        in_specs=(
            [pl.BlockSpec((e2, d), lambda i: (i, 0))
             for _ in range(len(Gparts))]
            + [pl.BlockSpec((bn, XK.shape[1]), lambda i: (i, 0)),
               pl.BlockSpec((bn, deg), lambda i: (i, 0))]
        ),
        out_specs=pl.BlockSpec((bn, d), lambda i: (i, 0)),
        out_shape=jax.ShapeDtypeStruct((n, d), jnp.float32),
    )(*Gparts, XK, nei_mask)


def kernel(x, nei, nei_mask, kernel_tangents, W, b):
    n, deg = nei.shape
    T, XK = _precompute(x, kernel_tangents, W, b)
    nei_flat = nei.reshape(-1).astype(jnp.int32)
    mask = nei_mask.astype(jnp.float32)
    ns = 5
    rows = n // ns
    outs = []
    for s in range(ns):
        lo = s * rows
        gparts = _sc_gather(
            lax.dynamic_slice_in_dim(nei_flat, lo * deg, rows * deg), T)
        outs.append(_aggregate(
            gparts,
            lax.dynamic_slice_in_dim(XK, lo, rows),
            lax.dynamic_slice_in_dim(mask, lo, rows)))
    return jnp.concatenate(outs, axis=0)
